# baseline (speedup 1.0000x reference)
import functools

import jax
import jax.numpy as jnp
from jax import lax
from jax.experimental import pallas as pl
from jax.experimental.pallas import tpu as pltpu
from jax.experimental.pallas import tpu_sc as plsc

N_MID = 1000000
DIM = 64
SEQ = 50
BATCH = 4096

NC = 2
NS = 16
NW = NC * NS

ROWS = BATCH * SEQ
ROWS_PER_W = ROWS // NW
G = 100
NG = 4
C = G * NG
CH = ROWS_PER_W // C
BATCHES_PER_CHUNK = C // SEQ


@functools.partial(
    pl.kernel,
    out_type=jax.ShapeDtypeStruct((ROWS, DIM), jnp.float32),
    mesh=plsc.VectorSubcoreMesh(core_axis_name="c", subcore_axis_name="s"),
    scratch_types=[
        pltpu.VMEM((NG, G), jnp.int32),
        pltpu.VMEM((C, DIM), jnp.float32),
        pltpu.VMEM((SEQ, DIM), jnp.float32),
        pltpu.SemaphoreType.DMA,
    ],
    compiler_params=pltpu.CompilerParams(use_tc_tiling_on_sc=False),
)
def _sc_lookup(idx_hbm, pos_hbm, table_hbm, out_hbm, idx_v, rows_v, pos_v, sem):
    cid = lax.axis_index("c")
    sid = lax.axis_index("s")
    wid = sid * NC + cid

    pltpu.sync_copy(pos_hbm, pos_v)

    def chunk_body(c, carry):
        pltpu.sync_copy(idx_hbm.at[wid, c], idx_v)
        copies = []
        for j in range(NG):
            copies.append(
                pltpu.async_copy(
                    table_hbm.at[idx_v.at[j]],
                    rows_v.at[pl.ds(j * G, G)],
                    sem,
                )
            )
        for cp in copies:
            cp.wait()

        def add_batch(b, carry2):
            def add_seq(s, carry3):
                r = b * SEQ + s
                for d in range(DIM // 16):
                    sl = pl.ds(d * 16, 16)
                    rows_v[r, sl] = rows_v[r, sl] + pos_v[s, sl]
                return carry3

            return lax.fori_loop(0, SEQ, add_seq, carry2)

        lax.fori_loop(0, BATCHES_PER_CHUNK, add_batch, carry)

        base = (wid * CH + c) * C
        pltpu.sync_copy(rows_v, out_hbm.at[pl.ds(base, C)])
        return carry

    lax.fori_loop(0, CH, chunk_body, 0)


def kernel(item, nbr_mask, i_ids, item_input_lookup, position_embedding):
    idx = item.reshape(NW, CH, NG, G)
    pos = position_embedding.reshape(SEQ, DIM)
    out = _sc_lookup(idx, pos, item_input_lookup)
    return out.reshape(BATCH, SEQ, DIM)


# speedup vs baseline: 1.1841x; 1.1841x over previous
import functools

import jax
import jax.numpy as jnp
from jax import lax
from jax.experimental import pallas as pl
from jax.experimental.pallas import tpu as pltpu
from jax.experimental.pallas import tpu_sc as plsc

N_MID = 1000000
DIM = 64
SEQ = 50
BATCH = 4096

NC = 2
NS = 16
NW = NC * NS

ROWS = BATCH * SEQ
ROWS_PER_W = ROWS // NW
G = 100
NG = 4
C = G * NG
CH = ROWS_PER_W // C
NB = C // SEQ
NLANE = DIM // 16


@functools.partial(
    pl.kernel,
    out_type=jax.ShapeDtypeStruct((ROWS, DIM), jnp.float32),
    mesh=plsc.VectorSubcoreMesh(core_axis_name="c", subcore_axis_name="s"),
    scratch_types=[
        pltpu.VMEM((2, NG, G), jnp.int32),
        pltpu.VMEM((2, C, DIM), jnp.float32),
        pltpu.VMEM((SEQ, DIM), jnp.float32),
        pltpu.SemaphoreType.DMA,
        pltpu.SemaphoreType.DMA,
        pltpu.SemaphoreType.DMA,
        pltpu.SemaphoreType.DMA,
    ],
    compiler_params=pltpu.CompilerParams(use_tc_tiling_on_sc=False),
)
def _sc_lookup(idx_hbm, pos_hbm, table_hbm, out_hbm,
               idx_v, buf, pos_v, gsem0, gsem1, ssem0, ssem1):
    cid = lax.axis_index("c")
    sid = lax.axis_index("s")
    wid = sid * NC + cid
    gsem = (gsem0, gsem1)
    ssem = (ssem0, ssem1)

    pltpu.sync_copy(pos_hbm, pos_v)

    def fire_chunk(t, par):
        pltpu.sync_copy(idx_hbm.at[wid, t], idx_v.at[par])
        for j in range(NG):
            pltpu.async_copy(
                table_hbm.at[idx_v.at[par, j]],
                buf.at[par, pl.ds(j * G, G)],
                gsem[par],
            )

    def drain_gather(par):
        pltpu.make_async_copy(
            out_hbm.at[pl.ds(0, C)], buf.at[par], gsem[par]
        ).wait()

    def drain_store(par):
        pltpu.make_async_copy(
            buf.at[par], out_hbm.at[pl.ds(0, C)], ssem[par]
        ).wait()

    fire_chunk(0, 0)

    def step(t, par):
        other = 1 - par

        @pl.when(t >= 1)
        def _():
            drain_store(other)

        @pl.when(t + 1 < CH)
        def _():
            fire_chunk(t + 1, other)

        drain_gather(par)

        def add_s(s, carry):
            pv = [pos_v[s, pl.ds(d * 16, 16)] for d in range(NLANE)]
            for b in range(NB):
                r = b * SEQ + s
                for d in range(NLANE):
                    sl = pl.ds(d * 16, 16)
                    buf[par, r, sl] = buf[par, r, sl] + pv[d]
            return carry

        lax.fori_loop(0, SEQ, add_s, 0)

        base = (wid * CH + t) * C
        pltpu.async_copy(buf.at[par], out_hbm.at[pl.ds(base, C)], ssem[par])

    def pair(tt, carry):
        step(tt * 2, 0)
        step(tt * 2 + 1, 1)
        return carry

    lax.fori_loop(0, CH // 2, pair, 0)
    drain_store((CH - 1) % 2)


def kernel(item, nbr_mask, i_ids, item_input_lookup, position_embedding):
    idx = item.reshape(NW, CH, NG, G)
    pos = position_embedding.reshape(SEQ, DIM)
    out = _sc_lookup(idx, pos, item_input_lookup)
    return out.reshape(BATCH, SEQ, DIM)


# speedup vs baseline: 1.7115x; 1.4454x over previous
import functools

import jax
import jax.numpy as jnp
from jax import lax
from jax.experimental import pallas as pl
from jax.experimental.pallas import tpu as pltpu
from jax.experimental.pallas import tpu_sc as plsc

N_MID = 1000000
DIM = 64
SEQ = 50
BATCH = 4096

NC = 2
NS = 16
NW = NC * NS

BPW = BATCH // NW
BPC = 4
C = BPC * SEQ
CH = BPW // BPC
NLANE = DIM // 16

GROUPS = ((0, 0, 16), (16, 0, 16), (32, 0, 16), (34, 14, 16))


@functools.partial(
    pl.kernel,
    out_type=jax.ShapeDtypeStruct((BATCH, SEQ, DIM), jnp.float32),
    mesh=plsc.VectorSubcoreMesh(core_axis_name="c", subcore_axis_name="s"),
    scratch_types=[
        pltpu.VMEM((C,), jnp.int32),
        pltpu.VMEM((C,), jnp.int32),
        pltpu.VMEM((2, BPC, SEQ, DIM), jnp.float32),
        pltpu.VMEM((SEQ, DIM), jnp.float32),
        pltpu.SemaphoreType.DMA,
        pltpu.SemaphoreType.DMA,
        pltpu.SemaphoreType.DMA,
        pltpu.SemaphoreType.DMA,
    ],
)
def _sc_lookup(item_hbm, pos_hbm, table_hbm, out_hbm,
               idx0, idx1, buf, pos_v, gsem0, gsem1, ssem0, ssem1):
    cid = lax.axis_index("c")
    sid = lax.axis_index("s")
    wid = sid * NC + cid
    idxs = (idx0, idx1)
    gsem = (gsem0, gsem1)
    ssem = (ssem0, ssem1)

    pltpu.sync_copy(pos_hbm, pos_v)

    def fire_chunk(t, par):
        base = wid * BPW * SEQ + t * C
        pltpu.sync_copy(item_hbm.at[pl.ds(base, C)], idxs[par])
        for b in range(BPC):
            for off, lane_lo, lane_hi in GROUPS:
                v = idxs[par][pl.ds(b * SEQ + off, 16)]
                for i in range(lane_lo, lane_hi):
                    row = v[i]
                    pltpu.async_copy(
                        table_hbm.at[pl.ds(row, 1)],
                        buf.at[par, b, pl.ds(off + i, 1)],
                        gsem[par],
                    )

    def drain_gather(par):
        pltpu.make_async_copy(
            out_hbm.at[pl.ds(0, BPC)], buf.at[par], gsem[par]
        ).wait()

    def drain_store(par):
        pltpu.make_async_copy(
            buf.at[par], out_hbm.at[pl.ds(0, BPC)], ssem[par]
        ).wait()

    fire_chunk(0, 0)

    def step(t, par):
        other = 1 - par

        @pl.when(t >= 1)
        def _():
            drain_store(other)

        @pl.when(t + 1 < CH)
        def _():
            fire_chunk(t + 1, other)

        drain_gather(par)

        def add_s(s, carry):
            pv = [pos_v[s, pl.ds(d * 16, 16)] for d in range(NLANE)]
            for b in range(BPC):
                for d in range(NLANE):
                    sl = pl.ds(d * 16, 16)
                    buf[par, b, s, sl] = buf[par, b, s, sl] + pv[d]
            return carry

        lax.fori_loop(0, SEQ, add_s, 0)

        bb = wid * BPW + t * BPC
        pltpu.async_copy(buf.at[par], out_hbm.at[pl.ds(bb, BPC)], ssem[par])

    def pair(tt, carry):
        step(tt * 2, 0)
        step(tt * 2 + 1, 1)
        return carry

    lax.fori_loop(0, CH // 2, pair, 0)
    drain_store((CH - 1) % 2)


def kernel(item, nbr_mask, i_ids, item_input_lookup, position_embedding):
    idx_flat = item.reshape(-1)
    pos = position_embedding.reshape(SEQ, DIM)
    return _sc_lookup(idx_flat, pos, item_input_lookup)
